# baseline (speedup 1.0000x reference)
import functools
import jax
import jax.numpy as jnp
from jax import lax
from jax.experimental import pallas as pl
from jax.experimental.pallas import tpu as pltpu
from jax.experimental.pallas import tpu_sc as plsc

D = 64
SCALE = 8.0
NC, NS = 2, 16
NW = NC * NS
V = 1000000
W = 128
S_LEN = 200
B_COLS = 4096
CV = 256
FULL_CHUNKS = V // CV
CPT = FULL_CHUNKS // NW
LEFTOVER = FULL_CHUNKS - CPT * NW
TAIL = V - FULL_CHUNKS * CV
SPH = FULL_CHUNKS * (CV // 2) + CV // 2

_mesh = lambda: plsc.VectorSubcoreMesh(core_axis_name="c", subcore_axis_name="s")


def _iota16():
    return lax.iota(jnp.int32, 16)


def _make_transpose():
    @functools.partial(
        pl.kernel,
        mesh=_mesh(),
        out_type=jax.ShapeDtypeStruct((SPH, W), jnp.float32),
        scratch_types=[
            pltpu.VMEM((D, CV), jnp.float32),
            pltpu.VMEM((D, CV), jnp.float32),
            pltpu.VMEM((D, CV), jnp.float32),
            pltpu.VMEM((128, W), jnp.float32),
            pltpu.VMEM((128, W), jnp.float32),
            pltpu.VMEM((128, W), jnp.float32),
            pltpu.SemaphoreType.DMA,
            pltpu.SemaphoreType.DMA,
            pltpu.SemaphoreType.DMA,
            pltpu.SemaphoreType.DMA,
            pltpu.SemaphoreType.DMA,
            pltpu.SemaphoreType.DMA,
        ],
        compiler_params=pltpu.CompilerParams(use_tc_tiling_on_sc=True, needs_layout_passes=False),
    )
    def tk(tab_t, tail_pad, tab_rm, in0, in1, in2, o0, o1, o2,
           g0, g1, g2, s0, s1, s2):
        wid = lax.axis_index("s") * NC + lax.axis_index("c")
        base = wid * CPT
        ins = (in0, in1, in2)
        outs = (o0, o1, o2)
        gsems = (g0, g1, g2)
        ssems = (s0, s1, s2)
        iota = _iota16()

        def v0_of(c):
            return (base + c) * CV

        def start_load(c, buf, sem):
            pltpu.async_copy(tab_t.at[:, pl.ds(v0_of(c), CV)], buf, sem)

        def wait_load(c, buf, sem):
            pltpu.make_async_copy(
                tab_t.at[:, pl.ds(v0_of(c), CV)], buf, sem
            ).wait()

        def r0_of(c):
            return (base + c) * (CV // 2)

        def start_store(c, buf, sem):
            pltpu.async_copy(buf, tab_rm.at[pl.ds(r0_of(c), CV // 2)], sem)

        def wait_store(c, buf, sem):
            pltpu.make_async_copy(
                buf, tab_rm.at[pl.ds(r0_of(c), CV // 2)], sem
            ).wait()

        halfl = lax.shift_right_logical(iota, 1)
        parl = lax.shift_left(iota & 1, 6)
        parl8 = lax.shift_left(iota & 1, 3)
        prows = [halfl + (8 * j) for j in range(8)]
        prows64 = [halfl + (64 + 8 * j) for j in range(8)]
        pskews = [((halfl + (8 * j)) & 15) + parl8 for j in range(8)]

        def transpose(ibuf, obuf, nhalves=2):
            def d_body(d, carry):
                cols = [parl + ((pskews[j] + d) & 63) for j in range(8)]
                vecs = [ibuf[d, pl.ds(j * 16, 16)] for j in range(8)]
                for j in range(8):
                    plsc.store_scatter(obuf, [prows[j], cols[j]], vecs[j])
                if nhalves == 2:
                    vecs2 = [ibuf[d, pl.ds(128 + j * 16, 16)] for j in range(8)]
                    for j in range(8):
                        plsc.store_scatter(obuf, [prows64[j], cols[j]], vecs2[j])
                return carry

            lax.fori_loop(0, D, d_body, 0, unroll=2)

        def step(c, q, with_wait_store, with_next_load):
            wait_load(c, ins[q], gsems[q])
            if with_wait_store:
                wait_store(c - 3, outs[q], ssems[q])
            transpose(ins[q], outs[q])
            start_store(c, outs[q], ssems[q])
            if with_next_load:
                start_load(c + 3, ins[q], gsems[q])

        start_load(0, ins[0], gsems[0])
        start_load(1, ins[1], gsems[1])
        start_load(2, ins[2], gsems[2])
        for c in range(3):
            step(c, c % 3, False, True)

        def outer(p, carry):
            g = 3 + 3 * p
            for q in range(3):
                step(g + q, q, True, True)
            return carry

        lax.fori_loop(0, (CPT - 8) // 3, outer, 0)

        for c in range(CPT - 5, CPT):
            step(c, c % 3, True, c + 3 < CPT)
        for c in range(CPT - 3, CPT):
            wait_store(c, outs[c % 3], ssems[c % 3])

        @pl.when(wid < LEFTOVER)
        def _():
            c = FULL_CHUNKS - LEFTOVER + wid
            pltpu.sync_copy(tab_t.at[:, pl.ds(c * CV, CV)], in0)
            transpose(in0, o0)
            pltpu.sync_copy(o0, tab_rm.at[pl.ds(c * (CV // 2), CV // 2)])

        @pl.when(wid == LEFTOVER)
        def _():
            pltpu.sync_copy(tail_pad, in0)
            transpose(in0, o0)
            pltpu.sync_copy(
                o0, tab_rm.at[pl.ds(FULL_CHUNKS * (CV // 2), CV // 2)]
            )

    return tk


def _make_gather():
    @functools.partial(
        pl.kernel,
        mesh=_mesh(),
        out_type=jax.ShapeDtypeStruct((S_LEN, D, B_COLS), jnp.float32),
        scratch_types=[
            pltpu.VMEM((S_LEN, 128), jnp.int32),
            pltpu.VMEM((S_LEN, 128), jnp.int32),
            pltpu.VMEM((128, W), jnp.float32),
            pltpu.VMEM((128, W), jnp.float32),
            pltpu.VMEM((128, W), jnp.float32),
            pltpu.VMEM((D, 128), jnp.float32),
            pltpu.VMEM((D, 128), jnp.float32),
            pltpu.VMEM((D, 128), jnp.float32),
            pltpu.SemaphoreType.DMA,
            pltpu.SemaphoreType.DMA,
            pltpu.SemaphoreType.DMA,
            pltpu.SemaphoreType.DMA,
            pltpu.SemaphoreType.DMA,
            pltpu.SemaphoreType.DMA,
        ],
        compiler_params=pltpu.CompilerParams(use_tc_tiling_on_sc=True, needs_layout_passes=False),
    )
    def gk(idx_t, tab_rm, out, idx_all, par64, r0, r1, r2, t0, t1, t2,
           g0, g1, g2, s0, s1, s2):
        wid = lax.axis_index("s") * NC + lax.axis_index("c")
        b0 = wid * 128
        rbufs = (r0, r1, r2)
        tbufs = (t0, t1, t2)
        gsems = (g0, g1, g2)
        ssems = (s0, s1, s2)
        iota = _iota16()

        pltpu.sync_copy(idx_t.at[:, pl.ds(b0, 128)], idx_all)

        def halve_body(s, carry):
            for j in range(8):
                sl = pl.ds(j * 16, 16)
                v = idx_all[s, sl]
                idx_all[s, sl] = lax.shift_right_logical(v, 1)
                par64[s, sl] = lax.shift_left(v & 1, 6)
            return carry

        lax.fori_loop(0, S_LEN, halve_body, 0, unroll=4)

        def start_gather(s, buf, sem):
            pltpu.async_copy(tab_rm.at[idx_all.at[s]], buf, sem)

        def wait_gather(s, buf, sem):
            pltpu.make_async_copy(tab_rm.at[idx_all.at[s]], buf, sem).wait()

        def start_store(s, buf, sem):
            pltpu.async_copy(buf, out.at[s, slice(None), pl.ds(b0, 128)], sem)

        def wait_store(s, buf, sem):
            pltpu.make_async_copy(
                buf, out.at[s, slice(None), pl.ds(b0, 128)], sem
            ).wait()

        rows = [iota + (j * 16) for j in range(8)]

        def transpose_scale(s, rbuf, tbuf):
            pars = [par64[s, pl.ds(j * 16, 16)] for j in range(8)]
            skews = [
                (idx_all[s, pl.ds(j * 16, 16)] & 15)
                + lax.shift_right_logical(pars[j], 3)
                for j in range(8)
            ]

            def d_body(d, carry):
                cols = [pars[j] + ((skews[j] + d) & 63) for j in range(8)]
                vecs = [
                    plsc.load_gather(rbuf, [rows[j], cols[j]])
                    for j in range(8)
                ]
                for j in range(8):
                    tbuf[d, pl.ds(j * 16, 16)] = vecs[j] * SCALE
                return carry

            lax.fori_loop(0, D, d_body, 0, unroll=4)

        def step(s, q, with_wait_store, with_next_gather):
            wait_gather(s, rbufs[q], gsems[q])
            if with_wait_store:
                wait_store(s - 3, tbufs[q], ssems[q])
            transpose_scale(s, rbufs[q], tbufs[q])
            start_store(s, tbufs[q], ssems[q])
            if with_next_gather:
                start_gather(s + 3, rbufs[q], gsems[q])

        start_gather(0, rbufs[0], gsems[0])
        start_gather(1, rbufs[1], gsems[1])
        start_gather(2, rbufs[2], gsems[2])
        for s in range(3):
            step(s, s % 3, False, True)

        def outer(p, carry):
            g = 3 + 3 * p
            for q in range(3):
                step(g + q, q, True, True)
            return carry

        lax.fori_loop(0, 64, outer, 0)

        for s in range(195, 200):
            step(s, s % 3, True, s + 3 < S_LEN)
        for s in range(197, 200):
            wait_store(s, tbufs[s % 3], ssems[s % 3])

    return gk


def kernel(input_tensor, table):
    idx_t = input_tensor.T
    tab_t = table.T
    tail_pad = jnp.pad(tab_t[:, FULL_CHUNKS * CV :], ((0, 0), (0, CV - TAIL)))
    tab_rm = _make_transpose()(tab_t, tail_pad)
    out_t = _make_gather()(idx_t, tab_rm)
    return out_t.transpose(2, 0, 1)


# speedup vs baseline: 1.0049x; 1.0049x over previous
import functools
import jax
import jax.numpy as jnp
from jax import lax
from jax.experimental import pallas as pl
from jax.experimental.pallas import tpu as pltpu
from jax.experimental.pallas import tpu_sc as plsc

D = 64
SCALE = 8.0
NC, NS = 2, 16
NW = NC * NS
V = 1000000
W = 128
S_LEN = 200
B_COLS = 4096
CV = 256
FULL_CHUNKS = V // CV
CPT = FULL_CHUNKS // NW
LEFTOVER = FULL_CHUNKS - CPT * NW
TAIL = V - FULL_CHUNKS * CV
SPH = FULL_CHUNKS * (CV // 2) + CV // 2

_mesh = lambda: plsc.VectorSubcoreMesh(core_axis_name="c", subcore_axis_name="s")


def _iota16():
    return lax.iota(jnp.int32, 16)


def _make_transpose():
    @functools.partial(
        pl.kernel,
        mesh=_mesh(),
        out_type=jax.ShapeDtypeStruct((SPH, W), jnp.float32),
        scratch_types=[
            pltpu.VMEM((D, CV), jnp.float32),
            pltpu.VMEM((D, CV), jnp.float32),
            pltpu.VMEM((D, CV), jnp.float32),
            pltpu.VMEM((128, W), jnp.float32),
            pltpu.VMEM((128, W), jnp.float32),
            pltpu.VMEM((128, W), jnp.float32),
            pltpu.SemaphoreType.DMA,
            pltpu.SemaphoreType.DMA,
            pltpu.SemaphoreType.DMA,
            pltpu.SemaphoreType.DMA,
            pltpu.SemaphoreType.DMA,
            pltpu.SemaphoreType.DMA,
        ],
        compiler_params=pltpu.CompilerParams(use_tc_tiling_on_sc=True, needs_layout_passes=False),
    )
    def tk(tab_t, tail_pad, tab_rm, in0, in1, in2, o0, o1, o2,
           g0, g1, g2, s0, s1, s2):
        wid = lax.axis_index("s") * NC + lax.axis_index("c")
        base = wid * CPT
        ins = (in0, in1, in2)
        outs = (o0, o1, o2)
        gsems = (g0, g1, g2)
        ssems = (s0, s1, s2)
        iota = _iota16()

        def v0_of(c):
            return (base + c) * CV

        def start_load(c, buf, sem):
            pltpu.async_copy(tab_t.at[:, pl.ds(v0_of(c), CV)], buf, sem)

        def wait_load(c, buf, sem):
            pltpu.make_async_copy(
                tab_t.at[:, pl.ds(v0_of(c), CV)], buf, sem
            ).wait()

        def r0_of(c):
            return (base + c) * (CV // 2)

        def start_store(c, buf, sem):
            pltpu.async_copy(buf, tab_rm.at[pl.ds(r0_of(c), CV // 2)], sem)

        def wait_store(c, buf, sem):
            pltpu.make_async_copy(
                buf, tab_rm.at[pl.ds(r0_of(c), CV // 2)], sem
            ).wait()

        halfl = lax.shift_right_logical(iota, 1)
        parl = lax.shift_left(iota & 1, 6)
        parl8 = lax.shift_left(iota & 1, 3)
        prows = [halfl + (8 * j) for j in range(8)]
        prows64 = [halfl + (64 + 8 * j) for j in range(8)]
        pskews = [((halfl + (8 * j)) & 15) + parl8 for j in range(8)]

        def transpose(ibuf, obuf, nhalves=2):
            def d_body(d, carry):
                cols = [parl + ((pskews[j] + d) & 63) for j in range(8)]
                vecs = [ibuf[d, pl.ds(j * 16, 16)] for j in range(8)]
                for j in range(8):
                    plsc.store_scatter(obuf, [prows[j], cols[j]], vecs[j])
                if nhalves == 2:
                    vecs2 = [ibuf[d, pl.ds(128 + j * 16, 16)] for j in range(8)]
                    for j in range(8):
                        plsc.store_scatter(obuf, [prows64[j], cols[j]], vecs2[j])
                return carry

            lax.fori_loop(0, D, d_body, 0, unroll=2)

        def step(c, q, with_wait_store, with_next_load):
            wait_load(c, ins[q], gsems[q])
            if with_wait_store:
                wait_store(c - 3, outs[q], ssems[q])
            transpose(ins[q], outs[q])
            start_store(c, outs[q], ssems[q])
            if with_next_load:
                start_load(c + 3, ins[q], gsems[q])

        start_load(0, ins[0], gsems[0])
        start_load(1, ins[1], gsems[1])
        start_load(2, ins[2], gsems[2])
        for c in range(3):
            step(c, c % 3, False, True)

        def outer(p, carry):
            g = 3 + 3 * p
            for q in range(3):
                step(g + q, q, True, True)
            return carry

        lax.fori_loop(0, (CPT - 8) // 3, outer, 0)

        for c in range(CPT - 5, CPT):
            step(c, c % 3, True, c + 3 < CPT)
        for c in range(CPT - 3, CPT):
            wait_store(c, outs[c % 3], ssems[c % 3])

        @pl.when(wid < LEFTOVER)
        def _():
            c = FULL_CHUNKS - LEFTOVER + wid
            pltpu.sync_copy(tab_t.at[:, pl.ds(c * CV, CV)], in0)
            transpose(in0, o0)
            pltpu.sync_copy(o0, tab_rm.at[pl.ds(c * (CV // 2), CV // 2)])

        @pl.when(wid == LEFTOVER)
        def _():
            pltpu.sync_copy(tail_pad, in0)
            transpose(in0, o0)
            pltpu.sync_copy(
                o0, tab_rm.at[pl.ds(FULL_CHUNKS * (CV // 2), CV // 2)]
            )

    return tk


def _make_gather():
    @functools.partial(
        pl.kernel,
        mesh=_mesh(),
        out_type=jax.ShapeDtypeStruct((S_LEN, D, B_COLS), jnp.float32),
        scratch_types=[
            pltpu.VMEM((S_LEN, 128), jnp.int32),
            pltpu.VMEM((S_LEN, 128), jnp.int32),
            pltpu.VMEM((128, W), jnp.float32),
            pltpu.VMEM((128, W), jnp.float32),
            pltpu.VMEM((128, W), jnp.float32),
            pltpu.VMEM((D, 128), jnp.float32),
            pltpu.VMEM((D, 128), jnp.float32),
            pltpu.VMEM((D, 128), jnp.float32),
            pltpu.SemaphoreType.DMA,
            pltpu.SemaphoreType.DMA,
            pltpu.SemaphoreType.DMA,
            pltpu.SemaphoreType.DMA,
            pltpu.SemaphoreType.DMA,
            pltpu.SemaphoreType.DMA,
        ],
        compiler_params=pltpu.CompilerParams(use_tc_tiling_on_sc=True, needs_layout_passes=False),
    )
    def gk(idx_t, tab_rm, gate, out, idx_all, par64, r0, r1, r2, t0, t1, t2,
           g0, g1, g2, s0, s1, s2):
        del gate
        wid = lax.axis_index("s") * NC + lax.axis_index("c")
        b0 = wid * 128
        rbufs = (r0, r1, r2)
        tbufs = (t0, t1, t2)
        gsems = (g0, g1, g2)
        ssems = (s0, s1, s2)
        iota = _iota16()

        pltpu.sync_copy(idx_t.at[:, pl.ds(b0, 128)], idx_all)

        def halve_body(s, carry):
            for j in range(8):
                sl = pl.ds(j * 16, 16)
                v = idx_all[s, sl]
                idx_all[s, sl] = lax.shift_right_logical(v, 1)
                par64[s, sl] = lax.shift_left(v & 1, 6)
            return carry

        lax.fori_loop(0, S_LEN, halve_body, 0, unroll=4)

        def start_gather(s, buf, sem):
            pltpu.async_copy(tab_rm.at[idx_all.at[s]], buf, sem)

        def wait_gather(s, buf, sem):
            pltpu.make_async_copy(tab_rm.at[idx_all.at[s]], buf, sem).wait()

        def start_store(s, buf, sem):
            pltpu.async_copy(buf, out.at[s, slice(None), pl.ds(b0, 128)], sem)

        def wait_store(s, buf, sem):
            pltpu.make_async_copy(
                buf, out.at[s, slice(None), pl.ds(b0, 128)], sem
            ).wait()

        rows = [iota + (j * 16) for j in range(8)]

        def transpose_scale(s, rbuf, tbuf):
            pars = [par64[s, pl.ds(j * 16, 16)] for j in range(8)]
            skews = [
                (idx_all[s, pl.ds(j * 16, 16)] & 15)
                + lax.shift_right_logical(pars[j], 3)
                for j in range(8)
            ]

            def d_body(d, carry):
                cols = [pars[j] + ((skews[j] + d) & 63) for j in range(8)]
                vecs = [
                    plsc.load_gather(rbuf, [rows[j], cols[j]])
                    for j in range(8)
                ]
                for j in range(8):
                    tbuf[d, pl.ds(j * 16, 16)] = vecs[j] * SCALE
                return carry

            lax.fori_loop(0, D, d_body, 0, unroll=4)

        def step(s, q, with_wait_store, with_next_gather):
            wait_gather(s, rbufs[q], gsems[q])
            if with_wait_store:
                wait_store(s - 3, tbufs[q], ssems[q])
            transpose_scale(s, rbufs[q], tbufs[q])
            start_store(s, tbufs[q], ssems[q])
            if with_next_gather:
                start_gather(s + 3, rbufs[q], gsems[q])

        start_gather(0, rbufs[0], gsems[0])
        start_gather(1, rbufs[1], gsems[1])
        start_gather(2, rbufs[2], gsems[2])
        for s in range(3):
            step(s, s % 3, False, True)

        def outer(p, carry):
            g = 3 + 3 * p
            for q in range(3):
                step(g + q, q, True, True)
            return carry

        lax.fori_loop(0, 64, outer, 0)

        for s in range(195, 200):
            step(s, s % 3, True, s + 3 < S_LEN)
        for s in range(197, 200):
            wait_store(s, tbufs[s % 3], ssems[s % 3])

    return gk


def kernel(input_tensor, table):
    idx_t = input_tensor.T
    tab_t = table.T
    tail_pad = jnp.pad(tab_t[:, FULL_CHUNKS * CV :], ((0, 0), (0, CV - TAIL)))
    tab_rm = _make_transpose()(tab_t, tail_pad)
    gate = tab_rm[:8, :128] + 0.0
    out_t = _make_gather()(idx_t, tab_rm, gate)
    return out_t.transpose(2, 0, 1)
